# Initial kernel scaffold; baseline (speedup 1.0000x reference)
import jax
import jax.numpy as jnp
from jax.experimental import pallas as pl

IMG_SZ = 384
NCLS = 80
NANCH = 3
KCAP = 1024
CONF_THR = 0.25
IOU_THR = 0.45
CLS_OFF = 4096.0


def _mm_bias_kernel(a_ref, w_ref, b_ref, o_ref):
    acc = jnp.dot(a_ref[...], w_ref[...], preferred_element_type=jnp.float32)
    o_ref[...] = acc + b_ref[...]


def _mm_bias(a, w, b, bm):
    m, k = a.shape
    n = w.shape[1]
    grid = (m // bm,)
    return pl.pallas_call(
        _mm_bias_kernel,
        grid=grid,
        in_specs=[
            pl.BlockSpec((bm, k), lambda i: (i, 0)),
            pl.BlockSpec((k, n), lambda i: (0, 0)),
            pl.BlockSpec((1, n), lambda i: (0, 0)),
        ],
        out_specs=pl.BlockSpec((bm, n), lambda i: (i, 0)),
        out_shape=jax.ShapeDtypeStruct((m, n), jnp.float32),
    )(a, w, b)


def _im2col_s2(x_nhwc):
    bsz, h, w, c = x_nhwc.shape
    ho, wo = h // 2, w // 2
    xp = jnp.pad(x_nhwc, ((0, 0), (0, 1), (0, 1), (0, 0)))
    slabs = []
    for dh in range(3):
        for dw in range(3):
            sl = xp[:, dh:dh + 2 * ho - 1:2, dw:dw + 2 * wo - 1:2, :]
            slabs.append(sl.reshape(bsz * ho * wo, c))
    return jnp.concatenate(slabs, axis=1)


def _wmat(w_oihw):
    co, ci = w_oihw.shape[0], w_oihw.shape[1]
    return w_oihw.transpose(2, 3, 1, 0).reshape(9 * ci, co)


def _nms_kernel(x1_ref, y1_ref, x2_ref, y2_ref, keep_ref):
    nimg = x1_ref.shape[0]
    idx = jax.lax.broadcasted_iota(jnp.int32, (KCAP,), 0)
    for img in range(nimg):
        x1 = x1_ref[img, :]
        y1 = y1_ref[img, :]
        x2 = x2_ref[img, :]
        y2 = y2_ref[img, :]
        area = (x2 - x1) * (y2 - y1)
        keep_ref[img, :] = jnp.ones((KCAP,), jnp.int32)

        def step(i, _):
            ki = keep_ref[img, pl.ds(i, 1)] > 0
            xi1 = jax.lax.dynamic_slice(x1, (i,), (1,))
            yi1 = jax.lax.dynamic_slice(y1, (i,), (1,))
            xi2 = jax.lax.dynamic_slice(x2, (i,), (1,))
            yi2 = jax.lax.dynamic_slice(y2, (i,), (1,))
            ai = jax.lax.dynamic_slice(area, (i,), (1,))
            ltx = jnp.maximum(xi1, x1)
            lty = jnp.maximum(yi1, y1)
            rbx = jnp.minimum(xi2, x2)
            rby = jnp.minimum(yi2, y2)
            iw = jnp.maximum(rbx - ltx, 0.0)
            ih = jnp.maximum(rby - lty, 0.0)
            inter = iw * ih
            iou = inter / (ai + area - inter + 1e-9)
            sup = (iou > IOU_THR) & (idx > i) & ki
            keep_ref[img, :] = jnp.where(sup, 0, keep_ref[img, :])
            return 0

        jax.lax.fori_loop(0, KCAP, step, 0)


def _nms_keep(x1, y1, x2, y2):
    nimg = x1.shape[0]
    return pl.pallas_call(
        _nms_kernel,
        out_shape=jax.ShapeDtypeStruct((nimg, KCAP), jnp.int32),
    )(x1, y1, x2, y2)


@jax.jit
def kernel(imgs, W_in, b_in, W_b1, b_b1, W_b2, b_b2, W_out, b_out):
    bsz = imgs.shape[0]
    bf = jnp.bfloat16

    x = imgs.transpose(0, 2, 3, 1)
    a1 = _im2col_s2(x).astype(bf)
    y1 = _mm_bias(a1, _wmat(W_in).astype(bf), b_in[None, :], bm=8192)
    x1 = jax.nn.silu(y1).reshape(bsz, 192, 192, 64)

    a2 = _im2col_s2(x1).astype(bf)
    y2 = _mm_bias(a2, _wmat(W_b1).astype(bf), b_b1[None, :], bm=2048)
    x2 = jax.nn.silu(y2).reshape(bsz, 96, 96, 128)

    a3 = _im2col_s2(x2).astype(bf)
    y3 = _mm_bias(a3, _wmat(W_b2).astype(bf), b_b2[None, :], bm=2304)
    x3 = jax.nn.silu(y3)

    w4 = W_out[:, :, 0, 0].T.astype(bf)
    p = _mm_bias(x3.astype(bf), w4, b_out[None, :], bm=2304)

    p3 = p.reshape(bsz, 2304, NANCH, NCLS + 5).transpose(0, 2, 1, 3)
    p3 = p3.reshape(bsz, NANCH * 2304, NCLS + 5)
    cx = jax.nn.sigmoid(p3[..., 0]) * float(IMG_SZ)
    cy = jax.nn.sigmoid(p3[..., 1]) * float(IMG_SZ)
    bw = jax.nn.sigmoid(p3[..., 2]) * float(IMG_SZ)
    bh = jax.nn.sigmoid(p3[..., 3]) * float(IMG_SZ)
    obj = jax.nn.sigmoid(p3[..., 4])
    cls = jax.nn.sigmoid(p3[..., 5:])
    conf = obj * jnp.max(cls, axis=-1)
    cls_id = jnp.argmax(cls, axis=-1)
    conf = jnp.where(conf > CONF_THR, conf, 0.0)
    bx1 = cx - bw / 2
    by1 = cy - bh / 2
    bx2 = cx + bw / 2
    by2 = cy + bh / 2

    _, idx = jax.lax.top_k(conf, KCAP)
    ss = jnp.take_along_axis(conf, idx, axis=1)
    sc = jnp.take_along_axis(cls_id, idx, axis=1)
    sx1 = jnp.take_along_axis(bx1, idx, axis=1)
    sy1 = jnp.take_along_axis(by1, idx, axis=1)
    sx2 = jnp.take_along_axis(bx2, idx, axis=1)
    sy2 = jnp.take_along_axis(by2, idx, axis=1)

    off = sc.astype(jnp.float32) * CLS_OFF
    keep = _nms_keep(sx1 + off, sy1 + off, sx2 + off, sy2 + off) > 0

    kf = keep.astype(jnp.float32)
    cx1 = jnp.clip(sx1, 0.0, float(IMG_SZ))
    cy1 = jnp.clip(sy1, 0.0, float(IMG_SZ))
    cx2 = jnp.clip(sx2, 0.0, float(IMG_SZ))
    cy2 = jnp.clip(sy2, 0.0, float(IMG_SZ))
    dets = jnp.stack([cx1 * kf, cy1 * kf, cx2 * kf, cy2 * kf, ss * kf], axis=2)
    labels = jnp.where(keep, sc, -1).astype(jnp.int32)
    return dets, labels


# speedup vs baseline: 3.0527x; 3.0527x over previous
import jax
import jax.numpy as jnp
from jax.experimental import pallas as pl

IMG_SZ = 384
NCLS = 80
NANCH = 3
KCAP = 1024
CONF_THR = 0.25
IOU_THR = 0.45
CLS_OFF = 4096.0


def _mm_bias_kernel(a_ref, w_ref, b_ref, o_ref):
    acc = jnp.dot(a_ref[...], w_ref[...], preferred_element_type=jnp.float32)
    o_ref[...] = acc + b_ref[...]


def _mm_bias(a, w, b, bm):
    m, k = a.shape
    n = w.shape[1]
    grid = (m // bm,)
    return pl.pallas_call(
        _mm_bias_kernel,
        grid=grid,
        in_specs=[
            pl.BlockSpec((bm, k), lambda i: (i, 0)),
            pl.BlockSpec((k, n), lambda i: (0, 0)),
            pl.BlockSpec((1, n), lambda i: (0, 0)),
        ],
        out_specs=pl.BlockSpec((bm, n), lambda i: (i, 0)),
        out_shape=jax.ShapeDtypeStruct((m, n), jnp.float32),
    )(a, w, b)


def _im2col_s2(x_nhwc):
    bsz, h, w, c = x_nhwc.shape
    ho, wo = h // 2, w // 2
    xp = jnp.pad(x_nhwc, ((0, 0), (0, 1), (0, 1), (0, 0)))
    slabs = []
    for dh in range(3):
        for dw in range(3):
            sl = xp[:, dh:dh + 2 * ho - 1:2, dw:dw + 2 * wo - 1:2, :]
            slabs.append(sl.reshape(bsz * ho * wo, c))
    return jnp.concatenate(slabs, axis=1)


def _wmat(w_oihw):
    co, ci = w_oihw.shape[0], w_oihw.shape[1]
    return w_oihw.transpose(2, 3, 1, 0).reshape(9 * ci, co)


def _nms_kernel(x1_ref, y1_ref, x2_ref, y2_ref,
                x1s_ref, y1s_ref, x2s_ref, y2s_ref, keep_ref):
    nimg = x1_ref.shape[0]
    idx = jax.lax.broadcasted_iota(jnp.int32, (KCAP,), 0)
    for img in range(nimg):
        x1 = x1_ref[img, :]
        y1 = y1_ref[img, :]
        x2 = x2_ref[img, :]
        y2 = y2_ref[img, :]
        area = (x2 - x1) * (y2 - y1)
        keep_ref[img, :] = jnp.ones((KCAP,), jnp.int32)

        def step(i, _):
            kvec = keep_ref[img, :]
            ki = jnp.sum(jnp.where(idx == i, kvec, 0)) > 0
            xi1 = x1s_ref[img, i]
            yi1 = y1s_ref[img, i]
            xi2 = x2s_ref[img, i]
            yi2 = y2s_ref[img, i]
            ai = (xi2 - xi1) * (yi2 - yi1)
            ltx = jnp.maximum(xi1, x1)
            lty = jnp.maximum(yi1, y1)
            rbx = jnp.minimum(xi2, x2)
            rby = jnp.minimum(yi2, y2)
            iw = jnp.maximum(rbx - ltx, 0.0)
            ih = jnp.maximum(rby - lty, 0.0)
            inter = iw * ih
            iou = inter / (ai + area - inter + 1e-9)
            sup = (iou > IOU_THR) & (idx > i) & ki
            keep_ref[img, :] = jnp.where(sup, 0, kvec)
            return 0

        jax.lax.fori_loop(0, KCAP, step, 0)


def _nms_keep(x1, y1, x2, y2):
    from jax.experimental.pallas import tpu as pltpu
    nimg = x1.shape[0]
    vspec = pl.BlockSpec((nimg, KCAP), lambda: (0, 0))
    sspec = pl.BlockSpec(memory_space=pltpu.SMEM)
    return pl.pallas_call(
        _nms_kernel,
        in_specs=[vspec, vspec, vspec, vspec, sspec, sspec, sspec, sspec],
        out_specs=pl.BlockSpec((nimg, KCAP), lambda: (0, 0)),
        out_shape=jax.ShapeDtypeStruct((nimg, KCAP), jnp.int32),
    )(x1, y1, x2, y2, x1, y1, x2, y2)


@jax.jit
def kernel(imgs, W_in, b_in, W_b1, b_b1, W_b2, b_b2, W_out, b_out):
    bsz = imgs.shape[0]
    bf = jnp.bfloat16

    x = imgs.transpose(0, 2, 3, 1)
    a1 = _im2col_s2(x).astype(bf)
    y1 = _mm_bias(a1, _wmat(W_in).astype(bf), b_in[None, :], bm=8192)
    x1 = jax.nn.silu(y1).reshape(bsz, 192, 192, 64)

    a2 = _im2col_s2(x1).astype(bf)
    y2 = _mm_bias(a2, _wmat(W_b1).astype(bf), b_b1[None, :], bm=2048)
    x2 = jax.nn.silu(y2).reshape(bsz, 96, 96, 128)

    a3 = _im2col_s2(x2).astype(bf)
    y3 = _mm_bias(a3, _wmat(W_b2).astype(bf), b_b2[None, :], bm=2304)
    x3 = jax.nn.silu(y3)

    w4 = W_out[:, :, 0, 0].T.astype(bf)
    p = _mm_bias(x3.astype(bf), w4, b_out[None, :], bm=2304)

    p3 = p.reshape(bsz, 2304, NANCH, NCLS + 5).transpose(0, 2, 1, 3)
    p3 = p3.reshape(bsz, NANCH * 2304, NCLS + 5)
    cx = jax.nn.sigmoid(p3[..., 0]) * float(IMG_SZ)
    cy = jax.nn.sigmoid(p3[..., 1]) * float(IMG_SZ)
    bw = jax.nn.sigmoid(p3[..., 2]) * float(IMG_SZ)
    bh = jax.nn.sigmoid(p3[..., 3]) * float(IMG_SZ)
    obj = jax.nn.sigmoid(p3[..., 4])
    cls = jax.nn.sigmoid(p3[..., 5:])
    conf = obj * jnp.max(cls, axis=-1)
    cls_id = jnp.argmax(cls, axis=-1)
    conf = jnp.where(conf > CONF_THR, conf, 0.0)
    bx1 = cx - bw / 2
    by1 = cy - bh / 2
    bx2 = cx + bw / 2
    by2 = cy + bh / 2

    _, idx = jax.lax.top_k(conf, KCAP)
    ss = jnp.take_along_axis(conf, idx, axis=1)
    sc = jnp.take_along_axis(cls_id, idx, axis=1)
    sx1 = jnp.take_along_axis(bx1, idx, axis=1)
    sy1 = jnp.take_along_axis(by1, idx, axis=1)
    sx2 = jnp.take_along_axis(bx2, idx, axis=1)
    sy2 = jnp.take_along_axis(by2, idx, axis=1)

    off = sc.astype(jnp.float32) * CLS_OFF
    keep = _nms_keep(sx1 + off, sy1 + off, sx2 + off, sy2 + off) > 0

    kf = keep.astype(jnp.float32)
    cx1 = jnp.clip(sx1, 0.0, float(IMG_SZ))
    cy1 = jnp.clip(sy1, 0.0, float(IMG_SZ))
    cx2 = jnp.clip(sx2, 0.0, float(IMG_SZ))
    cy2 = jnp.clip(sy2, 0.0, float(IMG_SZ))
    dets = jnp.stack([cx1 * kf, cy1 * kf, cx2 * kf, cy2 * kf, ss * kf], axis=2)
    labels = jnp.where(keep, sc, -1).astype(jnp.int32)
    return dets, labels


# speedup vs baseline: 3.3246x; 1.0890x over previous
import jax
import jax.numpy as jnp
from jax.experimental import pallas as pl

IMG_SZ = 384
NCLS = 80
NANCH = 3
KCAP = 1024
CONF_THR = 0.25
IOU_THR = 0.45
CLS_OFF = 4096.0


def _mm_bias_kernel(a_ref, w_ref, b_ref, o_ref):
    acc = jnp.dot(a_ref[...], w_ref[...], preferred_element_type=jnp.float32)
    o_ref[...] = acc + b_ref[...]


def _mm_bias(a, w, b, bm):
    m, k = a.shape
    n = w.shape[1]
    grid = (m // bm,)
    return pl.pallas_call(
        _mm_bias_kernel,
        grid=grid,
        in_specs=[
            pl.BlockSpec((bm, k), lambda i: (i, 0)),
            pl.BlockSpec((k, n), lambda i: (0, 0)),
            pl.BlockSpec((1, n), lambda i: (0, 0)),
        ],
        out_specs=pl.BlockSpec((bm, n), lambda i: (i, 0)),
        out_shape=jax.ShapeDtypeStruct((m, n), jnp.float32),
    )(a, w, b)


def _im2col_s2(x_nhwc):
    bsz, h, w, c = x_nhwc.shape
    ho, wo = h // 2, w // 2
    xp = jnp.pad(x_nhwc, ((0, 0), (0, 1), (0, 1), (0, 0)))
    slabs = []
    for dh in range(3):
        for dw in range(3):
            sl = xp[:, dh:dh + 2 * ho - 1:2, dw:dw + 2 * wo - 1:2, :]
            slabs.append(sl.reshape(bsz * ho * wo, c))
    return jnp.concatenate(slabs, axis=1)


def _wmat(w_oihw):
    co, ci = w_oihw.shape[0], w_oihw.shape[1]
    return w_oihw.transpose(2, 3, 1, 0).reshape(9 * ci, co)


def _nms_kernel(x1_ref, y1_ref, x2_ref, y2_ref,
                x1s_ref, y1s_ref, x2s_ref, y2s_ref, keep_ref):
    nimg = x1_ref.shape[0]
    idx = jax.lax.broadcasted_iota(jnp.int32, (nimg, KCAP), 1)
    x1 = x1_ref[...]
    y1 = y1_ref[...]
    x2 = x2_ref[...]
    y2 = y2_ref[...]
    area = (x2 - x1) * (y2 - y1)
    keep_ref[...] = jnp.ones((nimg, KCAP), jnp.int32)

    def col(sref, i):
        rows = [jnp.full((1, KCAP), sref[img, i], jnp.float32)
                for img in range(nimg)]
        return jnp.concatenate(rows, axis=0)

    def step(i, _):
        kvec = keep_ref[...]
        kirow = jnp.sum(jnp.where(idx == i, kvec, 0), axis=1, keepdims=True)
        xi1 = col(x1s_ref, i)
        yi1 = col(y1s_ref, i)
        xi2 = col(x2s_ref, i)
        yi2 = col(y2s_ref, i)
        ai = (xi2 - xi1) * (yi2 - yi1)
        ltx = jnp.maximum(xi1, x1)
        lty = jnp.maximum(yi1, y1)
        rbx = jnp.minimum(xi2, x2)
        rby = jnp.minimum(yi2, y2)
        iw = jnp.maximum(rbx - ltx, 0.0)
        ih = jnp.maximum(rby - lty, 0.0)
        inter = iw * ih
        iou = inter / (ai + area - inter + 1e-9)
        sup = (iou > IOU_THR) & (idx > i) & (kirow > 0)
        keep_ref[...] = jnp.where(sup, 0, kvec)
        return 0

    jax.lax.fori_loop(0, KCAP, step, 0)


def _nms_keep(x1, y1, x2, y2):
    from jax.experimental.pallas import tpu as pltpu
    nimg = x1.shape[0]
    vspec = pl.BlockSpec((nimg, KCAP), lambda: (0, 0))
    sspec = pl.BlockSpec(memory_space=pltpu.SMEM)
    return pl.pallas_call(
        _nms_kernel,
        in_specs=[vspec, vspec, vspec, vspec, sspec, sspec, sspec, sspec],
        out_specs=pl.BlockSpec((nimg, KCAP), lambda: (0, 0)),
        out_shape=jax.ShapeDtypeStruct((nimg, KCAP), jnp.int32),
    )(x1, y1, x2, y2, x1, y1, x2, y2)


@jax.jit
def kernel(imgs, W_in, b_in, W_b1, b_b1, W_b2, b_b2, W_out, b_out):
    bsz = imgs.shape[0]
    bf = jnp.bfloat16

    x = imgs.transpose(0, 2, 3, 1)
    a1 = _im2col_s2(x).astype(bf)
    y1 = _mm_bias(a1, _wmat(W_in).astype(bf), b_in[None, :], bm=8192)
    x1 = jax.nn.silu(y1).reshape(bsz, 192, 192, 64)

    a2 = _im2col_s2(x1).astype(bf)
    y2 = _mm_bias(a2, _wmat(W_b1).astype(bf), b_b1[None, :], bm=2048)
    x2 = jax.nn.silu(y2).reshape(bsz, 96, 96, 128)

    a3 = _im2col_s2(x2).astype(bf)
    y3 = _mm_bias(a3, _wmat(W_b2).astype(bf), b_b2[None, :], bm=2304)
    x3 = jax.nn.silu(y3)

    w4 = W_out[:, :, 0, 0].T.astype(bf)
    p = _mm_bias(x3.astype(bf), w4, b_out[None, :], bm=2304)

    p3 = p.reshape(bsz, 2304, NANCH, NCLS + 5).transpose(0, 2, 1, 3)
    p3 = p3.reshape(bsz, NANCH * 2304, NCLS + 5)
    cx = jax.nn.sigmoid(p3[..., 0]) * float(IMG_SZ)
    cy = jax.nn.sigmoid(p3[..., 1]) * float(IMG_SZ)
    bw = jax.nn.sigmoid(p3[..., 2]) * float(IMG_SZ)
    bh = jax.nn.sigmoid(p3[..., 3]) * float(IMG_SZ)
    obj = jax.nn.sigmoid(p3[..., 4])
    cls = jax.nn.sigmoid(p3[..., 5:])
    conf = obj * jnp.max(cls, axis=-1)
    cls_id = jnp.argmax(cls, axis=-1)
    conf = jnp.where(conf > CONF_THR, conf, 0.0)
    bx1 = cx - bw / 2
    by1 = cy - bh / 2
    bx2 = cx + bw / 2
    by2 = cy + bh / 2

    _, idx = jax.lax.top_k(conf, KCAP)
    ss = jnp.take_along_axis(conf, idx, axis=1)
    sc = jnp.take_along_axis(cls_id, idx, axis=1)
    sx1 = jnp.take_along_axis(bx1, idx, axis=1)
    sy1 = jnp.take_along_axis(by1, idx, axis=1)
    sx2 = jnp.take_along_axis(bx2, idx, axis=1)
    sy2 = jnp.take_along_axis(by2, idx, axis=1)

    off = sc.astype(jnp.float32) * CLS_OFF
    keep = _nms_keep(sx1 + off, sy1 + off, sx2 + off, sy2 + off) > 0

    kf = keep.astype(jnp.float32)
    cx1 = jnp.clip(sx1, 0.0, float(IMG_SZ))
    cy1 = jnp.clip(sy1, 0.0, float(IMG_SZ))
    cx2 = jnp.clip(sx2, 0.0, float(IMG_SZ))
    cy2 = jnp.clip(sy2, 0.0, float(IMG_SZ))
    dets = jnp.stack([cx1 * kf, cy1 * kf, cx2 * kf, cy2 * kf, ss * kf], axis=2)
    labels = jnp.where(keep, sc, -1).astype(jnp.int32)
    return dets, labels
